# Initial kernel scaffold; baseline (speedup 1.0000x reference)
import jax
import jax.numpy as jnp
from jax.experimental import pallas as pl


def kernel(x, edge_index, edge_attr, batch, W0, b0, We1, be1, We2, be2, Wroot, bconv, Wih, Whh, bih, bhh, Wih_s, Whh_s, bih_s, bhh_s, W1, b1, W2, b2, W3, b3):
    raise NotImplementedError("write your pallas kernel here")


# speedup vs baseline: 1.9735x; 1.9735x over previous
import functools

import jax
import jax.numpy as jnp
import numpy as np
from jax import lax
from jax.experimental import pallas as pl
from jax.experimental.pallas import tpu as pltpu
from jax.experimental.pallas import tpu_sc as plsc

N = 10000
E = 160000
NF = 128
H = 32
D = 32
B = 512

NP_ = 10240
EP = 163840
NW = 32
EPW = EP // NW
CPW = EPW // 128
NB = 256
EB = 512
NPS = NP_ // 16

_f32 = jnp.float32


def _sc_mesh():
    return plsc.VectorSubcoreMesh(core_axis_name="c", subcore_axis_name="s")


_SC_PARAMS = pltpu.CompilerParams(use_tc_tiling_on_sc=False)


def _sc_gather(src2d, table):
    @functools.partial(
        pl.kernel,
        out_type=jax.ShapeDtypeStruct((EP, H), _f32),
        mesh=_sc_mesh(),
        compiler_params=_SC_PARAMS,
        scratch_types=[
            pltpu.VMEM((CPW, 128), jnp.int32),
            pltpu.VMEM((2560, H), _f32),
            pltpu.SemaphoreType.DMA,
        ],
    )
    def k(src_hbm, tab_hbm, out_hbm, idx_v, rows_v, sem):
        c = lax.axis_index("c")
        s = lax.axis_index("s")
        wid = c * 16 + s
        pltpu.sync_copy(src_hbm.at[pl.ds(wid * CPW, CPW), :], idx_v)
        for half in range(2):
            cps = [
                pltpu.async_copy(
                    tab_hbm.at[idx_v.at[half * 20 + j]],
                    rows_v.at[pl.ds(j * 128, 128), :],
                    sem,
                )
                for j in range(20)
            ]
            for cp in cps:
                cp.wait()
            pltpu.sync_copy(
                rows_v, out_hbm.at[pl.ds(wid * EPW + half * 2560, 2560), :]
            )

    return k(src2d, table)


def _sc_scatter(dst2d, msg, zeros32):
    @functools.partial(
        pl.kernel,
        out_type=jax.ShapeDtypeStruct((2, NP_, H), _f32),
        mesh=_sc_mesh(),
        compiler_params=_SC_PARAMS,
        scratch_types=[
            pltpu.VMEM((CPW, 128), jnp.int32),
            pltpu.VMEM((2560, H), _f32),
            pltpu.VMEM_SHARED((NP_, H), _f32),
        ],
    )
    def k(dst_hbm, msg_hbm, z_hbm, out_hbm, idx_v, rows_v, acc_sh):
        c = lax.axis_index("c")
        s = lax.axis_index("s")
        wid = c * 16 + s
        pltpu.sync_copy(z_hbm, rows_v.at[pl.ds(0, NPS), :])
        pltpu.sync_copy(rows_v.at[pl.ds(0, NPS), :], acc_sh.at[pl.ds(s * NPS, NPS), :])
        plsc.subcore_barrier()
        pltpu.sync_copy(dst_hbm.at[pl.ds(wid * CPW, CPW), :], idx_v)
        for half in range(2):
            pltpu.sync_copy(
                msg_hbm.at[pl.ds(wid * EPW + half * 2560, 2560), :], rows_v
            )
            for j in range(20):
                pltpu.sync_copy(
                    rows_v.at[pl.ds(j * 128, 128), :],
                    acc_sh.at[idx_v.at[half * 20 + j]],
                    add=True,
                )
        plsc.subcore_barrier()
        pltpu.sync_copy(acc_sh.at[pl.ds(s * NPS, NPS), :], rows_v.at[pl.ds(0, NPS), :])
        pltpu.sync_copy(rows_v.at[pl.ds(0, NPS), :], out_hbm.at[c, pl.ds(s * NPS, NPS), :])

    return k(dst2d, msg, zeros32)


def _sc_deg(dst2d, ones16, zeros16):
    @functools.partial(
        pl.kernel,
        out_type=jax.ShapeDtypeStruct((2, NP_, 16), _f32),
        mesh=_sc_mesh(),
        compiler_params=_SC_PARAMS,
        scratch_types=[
            pltpu.VMEM((CPW, 128), jnp.int32),
            pltpu.VMEM((128, 16), _f32),
            pltpu.VMEM((NPS, 16), _f32),
            pltpu.VMEM_SHARED((NP_, 16), _f32),
        ],
    )
    def k(dst_hbm, ones_hbm, z_hbm, out_hbm, idx_v, ones_v, buf_v, acc_sh):
        c = lax.axis_index("c")
        s = lax.axis_index("s")
        wid = c * 16 + s
        pltpu.sync_copy(z_hbm, buf_v)
        pltpu.sync_copy(buf_v, acc_sh.at[pl.ds(s * NPS, NPS), :])
        plsc.subcore_barrier()
        pltpu.sync_copy(ones_hbm, ones_v)
        pltpu.sync_copy(dst_hbm.at[pl.ds(wid * CPW, CPW), :], idx_v)
        for j in range(CPW):
            pltpu.sync_copy(ones_v, acc_sh.at[idx_v.at[j]], add=True)
        plsc.subcore_barrier()
        pltpu.sync_copy(acc_sh.at[pl.ds(s * NPS, NPS), :], buf_v)
        pltpu.sync_copy(buf_v, out_hbm.at[c, pl.ds(s * NPS, NPS), :])

    return k(dst2d, ones16, zeros16)


def _full(shape):
    nd = len(shape)
    return pl.BlockSpec(shape, lambda i: (0,) * nd)


def _out0_body(x_ref, w_ref, b_ref, o_ref):
    o_ref[...] = jnp.maximum(x_ref[...] @ w_ref[...] + b_ref[...], 0.0)


def _tc_out0(xp, W0T, b0r):
    return pl.pallas_call(
        _out0_body,
        grid=(NP_ // NB,),
        in_specs=[
            pl.BlockSpec((NB, NF), lambda i: (i, 0)),
            _full((NF, H)),
            _full((1, H)),
        ],
        out_specs=pl.BlockSpec((NB, H), lambda i: (i, 0)),
        out_shape=jax.ShapeDtypeStruct((NP_, H), _f32),
    )(xp, W0T, b0r)


def _edge_body(xj_ref, ea_ref, w1_ref, b1_ref, w2_ref, b2_ref, r_ref, s_ref, o_ref):
    r = jnp.maximum(ea_ref[...] @ w1_ref[...] + b1_ref[...], 0.0)
    ew = r @ w2_ref[...] + b2_ref[...]
    xb = xj_ref[...] @ r_ref[...]
    o_ref[...] = (xb * ew) @ s_ref[...]


def _tc_edge(xj, eap, We1p, be1r, We2T, be2r, Rm, Sm):
    return pl.pallas_call(
        _edge_body,
        grid=(EP // EB,),
        in_specs=[
            pl.BlockSpec((EB, H), lambda i: (i, 0)),
            pl.BlockSpec((EB, 8), lambda i: (i, 0)),
            _full((8, 128)),
            _full((1, 128)),
            _full((128, H * H)),
            _full((1, H * H)),
            _full((H, H * H)),
            _full((H * H, H)),
        ],
        out_specs=pl.BlockSpec((EB, H), lambda i: (i, 0)),
        out_shape=jax.ShapeDtypeStruct((EP, H), _f32),
    )(xj, eap, We1p, be1r, We2T, be2r, Rm, Sm)


def _node_body(ap_ref, dp_ref, s_ref, wr_ref, bc_ref, wi_ref, bi_ref, wh_ref, bh_ref, o_ref):
    deg = jnp.maximum(dp_ref[0, :, 0:1] + dp_ref[1, :, 0:1], 1.0)
    agg = (ap_ref[0] + ap_ref[1]) / deg
    s = s_ref[...]
    m = jnp.maximum(agg + s @ wr_ref[...] + bc_ref[...], 0.0)
    gi = m @ wi_ref[...] + bi_ref[...]
    gh = s @ wh_ref[...] + bh_ref[...]
    r = jax.nn.sigmoid(gi[:, 0:H] + gh[:, 0:H])
    z = jax.nn.sigmoid(gi[:, H:2 * H] + gh[:, H:2 * H])
    n = jnp.tanh(gi[:, 2 * H:3 * H] + r * gh[:, 2 * H:3 * H])
    o_ref[...] = (1.0 - z) * n + z * s


def _tc_node(aggp, degp, s, WrootM, bconvr, WihT, bihr, WhhT, bhhr):
    return pl.pallas_call(
        _node_body,
        grid=(NP_ // NB,),
        in_specs=[
            pl.BlockSpec((2, NB, H), lambda i: (0, i, 0)),
            pl.BlockSpec((2, NB, 16), lambda i: (0, i, 0)),
            pl.BlockSpec((NB, H), lambda i: (i, 0)),
            _full((H, H)),
            _full((1, H)),
            _full((H, 3 * H)),
            _full((1, 3 * H)),
            _full((H, 3 * H)),
            _full((1, 3 * H)),
        ],
        out_specs=pl.BlockSpec((NB, H), lambda i: (i, 0)),
        out_shape=jax.ShapeDtypeStruct((NP_, H), _f32),
    )(aggp, degp, s, WrootM, bconvr, WihT, bihr, WhhT, bhhr)


def _ne_body(s_ref, w_ref, b_ref, o_ref):
    o_ref[...] = s_ref[...] @ w_ref[...] + b_ref[...]


def _tc_ne(s, W1T, b1r):
    return pl.pallas_call(
        _ne_body,
        grid=(NP_ // NB,),
        in_specs=[
            pl.BlockSpec((NB, H), lambda i: (i, 0)),
            _full((H, D)),
            _full((1, D)),
        ],
        out_specs=pl.BlockSpec((NB, D), lambda i: (i, 0)),
        out_shape=jax.ShapeDtypeStruct((NP_, D), _f32),
    )(s, W1T, b1r)


def _lstm_body(q_ref, rv_ref, hs_ref, cs_ref, wi_ref, bi_ref, wh_ref, bh_ref, ho_ref, co_ref):
    rvec = rv_ref[:, 0:D] / (rv_ref[:, D:D + 1] + 1e-16)
    q_star = jnp.concatenate([q_ref[...], rvec], axis=1)
    gates = q_star @ wi_ref[...] + bi_ref[...] + hs_ref[...] @ wh_ref[...] + bh_ref[...]
    gi = gates[:, 0:D]
    gf = gates[:, D:2 * D]
    gg = gates[:, 2 * D:3 * D]
    go = gates[:, 3 * D:4 * D]
    cs = jax.nn.sigmoid(gf) * cs_ref[...] + jax.nn.sigmoid(gi) * jnp.tanh(gg)
    ho_ref[...] = jax.nn.sigmoid(go) * jnp.tanh(cs)
    co_ref[...] = cs


def _tc_lstm(qprev, rvacc, hs, cs, WihsT, bihsr, WhhsT, bhhsr):
    return pl.pallas_call(
        _lstm_body,
        out_shape=(
            jax.ShapeDtypeStruct((B, D), _f32),
            jax.ShapeDtypeStruct((B, D), _f32),
        ),
    )(qprev, rvacc, hs, cs, WihsT, bihsr, WhhsT, bhhsr)


def _pass1_body(ne_ref, b_ref, q_ref, e_ref, mx_ref):
    pid = pl.program_id(0)
    cols = lax.broadcasted_iota(jnp.int32, (NB, B), 1)
    ohb = b_ref[...] == cols
    oh = ohb.astype(_f32)
    qg = oh @ q_ref[...]
    e = jnp.sum(ne_ref[...] * qg, axis=1, keepdims=True)
    e_ref[...] = e
    part = jnp.max(jnp.where(ohb, e, -1e30), axis=0, keepdims=True)

    @pl.when(pid == 0)
    def _():
        mx_ref[...] = jnp.full((1, B), -1e30, _f32)

    mx_ref[...] = jnp.maximum(mx_ref[...], part)


def _tc_pass1(nep, batchc, q):
    return pl.pallas_call(
        _pass1_body,
        grid=(NP_ // NB,),
        in_specs=[
            pl.BlockSpec((NB, D), lambda i: (i, 0)),
            pl.BlockSpec((NB, 1), lambda i: (i, 0)),
            _full((B, D)),
        ],
        out_specs=(
            pl.BlockSpec((NB, 1), lambda i: (i, 0)),
            pl.BlockSpec((1, B), lambda i: (0, 0)),
        ),
        out_shape=(
            jax.ShapeDtypeStruct((NP_, 1), _f32),
            jax.ShapeDtypeStruct((1, B), _f32),
        ),
    )(nep, batchc, q)


def _pass2_body(ne_ref, b_ref, e_ref, mx_ref, rv_ref):
    pid = pl.program_id(0)
    cols = lax.broadcasted_iota(jnp.int32, (NB, B), 1)
    oh = (b_ref[...] == cols).astype(_f32)
    mxg = jnp.sum(oh * mx_ref[...], axis=1, keepdims=True)
    a = jnp.exp(e_ref[...] - mxg)
    oa = oh * a
    ne_aug = jnp.concatenate([ne_ref[...], jnp.ones((NB, 1), _f32)], axis=1)
    part = lax.dot_general(oa, ne_aug, (((0,), (0,)), ((), ())),
                           preferred_element_type=_f32)

    @pl.when(pid == 0)
    def _():
        rv_ref[...] = jnp.zeros((B, D + 1), _f32)

    rv_ref[...] = rv_ref[...] + part


def _tc_pass2(nep, batchc, e_col, emax):
    return pl.pallas_call(
        _pass2_body,
        grid=(NP_ // NB,),
        in_specs=[
            pl.BlockSpec((NB, D), lambda i: (i, 0)),
            pl.BlockSpec((NB, 1), lambda i: (i, 0)),
            pl.BlockSpec((NB, 1), lambda i: (i, 0)),
            _full((1, B)),
        ],
        out_specs=pl.BlockSpec((B, D + 1), lambda i: (0, 0)),
        out_shape=jax.ShapeDtypeStruct((B, D + 1), _f32),
    )(nep, batchc, e_col, emax)


def _head_body(q_ref, rv_ref, w2_ref, b2_ref, w3_ref, b3_ref, ge_ref, pr_ref):
    rvec = rv_ref[:, 0:D] / (rv_ref[:, D:D + 1] + 1e-16)
    q_star = jnp.concatenate([q_ref[...], rvec], axis=1)
    ge = q_star @ w2_ref[...] + b2_ref[...]
    ge_ref[...] = ge
    pr_ref[...] = ge @ w3_ref[...] + b3_ref[...]


def _tc_head(qprev, rvacc, W2T, b2r, W3T, b3r):
    return pl.pallas_call(
        _head_body,
        out_shape=(
            jax.ShapeDtypeStruct((B, D), _f32),
            jax.ShapeDtypeStruct((B, 1), _f32),
        ),
    )(qprev, rvacc, W2T, b2r, W3T, b3r)


def kernel(x, edge_index, edge_attr, batch, W0, b0, We1, be1, We2, be2,
           Wroot, bconv, Wih, Whh, bih, bhh, Wih_s, Whh_s, bih_s, bhh_s,
           W1, b1, W2, b2, W3, b3):
    xp = jnp.pad(x, ((0, NP_ - N), (0, 0)))
    src2d = jnp.pad(edge_index[0], (0, EP - E)).reshape(EP // 128, 128)
    dst2d = jnp.pad(edge_index[1], (0, EP - E),
                    constant_values=NP_ - 1).reshape(EP // 128, 128)
    eap = jnp.pad(edge_attr, ((0, EP - E), (0, 5)))
    batchc = jnp.pad(batch, (0, NP_ - N), constant_values=B).reshape(NP_, 1)

    W0T = W0.T
    b0r = b0.reshape(1, -1)
    We1p = jnp.pad(We1.T, ((0, 5), (0, 0)))
    be1r = be1.reshape(1, -1)
    We2T = We2.T
    be2r = be2.reshape(1, -1)
    bconvr = bconv.reshape(1, -1)
    WihT = Wih.T
    bihr = bih.reshape(1, -1)
    WhhT = Whh.T
    bhhr = bhh.reshape(1, -1)
    WihsT = Wih_s.T
    bihsr = bih_s.reshape(1, -1)
    WhhsT = Whh_s.T
    bhhsr = bhh_s.reshape(1, -1)
    W1T = W1.T
    b1r = b1.reshape(1, -1)
    W2T = W2.T
    b2r = b2.reshape(1, -1)
    W3T = W3.T
    b3r = b3.reshape(1, -1)

    Rm = jnp.asarray(np.kron(np.eye(H, dtype=np.float32), np.ones((1, H), np.float32)))
    Sm = jnp.asarray(np.tile(np.eye(H, dtype=np.float32), (H, 1)))
    ones16 = jnp.ones((128, 16), _f32)
    zeros16 = jnp.zeros((NPS, 16), _f32)
    zeros32 = jnp.zeros((NPS, H), _f32)

    s = _tc_out0(xp, W0T, b0r)
    degp = _sc_deg(dst2d, ones16, zeros16)
    for _ in range(3):
        xj = _sc_gather(src2d, s)
        msg = _tc_edge(xj, eap, We1p, be1r, We2T, be2r, Rm, Sm)
        aggp = _sc_scatter(dst2d, msg, zeros32)
        s = _tc_node(aggp, degp, s, Wroot, bconvr, WihT, bihr, WhhT, bhhr)

    nep = _tc_ne(s, W1T, b1r)
    hs = jnp.zeros((B, D), _f32)
    cs = jnp.zeros((B, D), _f32)
    qprev = jnp.zeros((B, D), _f32)
    rvacc = jnp.zeros((B, D + 1), _f32)
    for _ in range(3):
        hs, cs = _tc_lstm(qprev, rvacc, hs, cs, WihsT, bihsr, WhhsT, bhhsr)
        qprev = hs
        e_col, emax = _tc_pass1(nep, batchc, hs)
        rvacc = _tc_pass2(nep, batchc, e_col, emax)

    ge, pred = _tc_head(qprev, rvacc, W2T, b2r, W3T, b3r)
    return pred.reshape(-1), ge, nep[:N]


# speedup vs baseline: 2.5655x; 1.3000x over previous
import functools

import jax
import jax.numpy as jnp
import numpy as np
from jax import lax
from jax.experimental import pallas as pl
from jax.experimental.pallas import tpu as pltpu
from jax.experimental.pallas import tpu_sc as plsc

N = 10000
E = 160000
NF = 128
H = 32
D = 32
B = 512

NP_ = 10240
EP = 163840
NW = 32
EPW = EP // NW
CPW = EPW // 128
NB = 256
EB = 1024
NPS = NP_ // 16

_f32 = jnp.float32


def _sc_mesh():
    return plsc.VectorSubcoreMesh(core_axis_name="c", subcore_axis_name="s")


_SC_PARAMS = pltpu.CompilerParams(use_tc_tiling_on_sc=False)


def _sc_gather(src2d, table):
    @functools.partial(
        pl.kernel,
        out_type=jax.ShapeDtypeStruct((EP, H), _f32),
        mesh=_sc_mesh(),
        compiler_params=_SC_PARAMS,
        scratch_types=[
            pltpu.VMEM((CPW, 128), jnp.int32),
            pltpu.VMEM((2560, H), _f32),
            pltpu.SemaphoreType.DMA,
        ],
    )
    def k(src_hbm, tab_hbm, out_hbm, idx_v, rows_v, sem):
        c = lax.axis_index("c")
        s = lax.axis_index("s")
        wid = c * 16 + s
        pltpu.sync_copy(src_hbm.at[pl.ds(wid * CPW, CPW), :], idx_v)
        for half in range(2):
            cps = [
                pltpu.async_copy(
                    tab_hbm.at[idx_v.at[half * 20 + j]],
                    rows_v.at[pl.ds(j * 128, 128), :],
                    sem,
                )
                for j in range(20)
            ]
            for cp in cps:
                cp.wait()
            pltpu.sync_copy(
                rows_v, out_hbm.at[pl.ds(wid * EPW + half * 2560, 2560), :]
            )

    return k(src2d, table)


def _sc_scatter(dst2d, msg, zeros32):
    @functools.partial(
        pl.kernel,
        out_type=jax.ShapeDtypeStruct((2, NP_, H), _f32),
        mesh=_sc_mesh(),
        compiler_params=_SC_PARAMS,
        scratch_types=[
            pltpu.VMEM((CPW, 128), jnp.int32),
            pltpu.VMEM((1280, H), _f32),
            pltpu.VMEM((1280, H), _f32),
            pltpu.VMEM_SHARED((NP_, H), _f32),
            pltpu.SemaphoreType.DMA,
            pltpu.SemaphoreType.DMA,
        ],
    )
    def k(dst_hbm, msg_hbm, z_hbm, out_hbm, idx_v, buf0_v, buf1_v, acc_sh,
          sem_l, sem_s):
        c = lax.axis_index("c")
        s = lax.axis_index("s")
        wid = c * 16 + s
        pltpu.sync_copy(z_hbm, buf0_v.at[pl.ds(0, NPS), :])
        pltpu.sync_copy(buf0_v.at[pl.ds(0, NPS), :], acc_sh.at[pl.ds(s * NPS, NPS), :])
        plsc.subcore_barrier()
        pltpu.sync_copy(dst_hbm.at[pl.ds(wid * CPW, CPW), :], idx_v)
        bufs = (buf0_v, buf1_v)
        scat = [[], []]
        loads = [None, None]
        loads[0] = pltpu.async_copy(
            msg_hbm.at[pl.ds(wid * EPW, 1280), :], buf0_v, sem_l)
        for q in range(4):
            b = q % 2
            loads[b].wait()
            scat[b] = [
                pltpu.async_copy(
                    bufs[b].at[pl.ds(j * 128, 128), :],
                    acc_sh.at[idx_v.at[q * 10 + j]],
                    sem_s,
                    add=True,
                )
                for j in range(10)
            ]
            if q < 3:
                nb = (q + 1) % 2
                for cp in scat[nb]:
                    cp.wait()
                scat[nb] = []
                loads[nb] = pltpu.async_copy(
                    msg_hbm.at[pl.ds(wid * EPW + (q + 1) * 1280, 1280), :],
                    bufs[nb], sem_l)
        for b in range(2):
            for cp in scat[b]:
                cp.wait()
        plsc.subcore_barrier()
        pltpu.sync_copy(acc_sh.at[pl.ds(s * NPS, NPS), :], buf0_v.at[pl.ds(0, NPS), :])
        pltpu.sync_copy(buf0_v.at[pl.ds(0, NPS), :], out_hbm.at[c, pl.ds(s * NPS, NPS), :])

    return k(dst2d, msg, zeros32)


def _sc_deg(dst2d, ones16, zeros16):
    @functools.partial(
        pl.kernel,
        out_type=jax.ShapeDtypeStruct((2, NP_, 16), _f32),
        mesh=_sc_mesh(),
        compiler_params=_SC_PARAMS,
        scratch_types=[
            pltpu.VMEM((CPW, 128), jnp.int32),
            pltpu.VMEM((128, 16), _f32),
            pltpu.VMEM((NPS, 16), _f32),
            pltpu.VMEM_SHARED((NP_, 16), _f32),
        ],
    )
    def k(dst_hbm, ones_hbm, z_hbm, out_hbm, idx_v, ones_v, buf_v, acc_sh):
        c = lax.axis_index("c")
        s = lax.axis_index("s")
        wid = c * 16 + s
        pltpu.sync_copy(z_hbm, buf_v)
        pltpu.sync_copy(buf_v, acc_sh.at[pl.ds(s * NPS, NPS), :])
        plsc.subcore_barrier()
        pltpu.sync_copy(ones_hbm, ones_v)
        pltpu.sync_copy(dst_hbm.at[pl.ds(wid * CPW, CPW), :], idx_v)
        for j in range(CPW):
            pltpu.sync_copy(ones_v, acc_sh.at[idx_v.at[j]], add=True)
        plsc.subcore_barrier()
        pltpu.sync_copy(acc_sh.at[pl.ds(s * NPS, NPS), :], buf_v)
        pltpu.sync_copy(buf_v, out_hbm.at[c, pl.ds(s * NPS, NPS), :])

    return k(dst2d, ones16, zeros16)


def _full(shape):
    nd = len(shape)
    return pl.BlockSpec(shape, lambda i: (0,) * nd)


def _out0_body(x_ref, w_ref, b_ref, o_ref):
    o_ref[...] = jnp.maximum(x_ref[...] @ w_ref[...] + b_ref[...], 0.0)


def _tc_out0(xp, W0T, b0r):
    return pl.pallas_call(
        _out0_body,
        grid=(NP_ // NB,),
        in_specs=[
            pl.BlockSpec((NB, NF), lambda i: (i, 0)),
            _full((NF, H)),
            _full((1, H)),
        ],
        out_specs=pl.BlockSpec((NB, H), lambda i: (i, 0)),
        out_shape=jax.ShapeDtypeStruct((NP_, H), _f32),
    )(xp, W0T, b0r)


def _edge_body(xj_ref, ea_ref, w1_ref, b1_ref, w2_ref, b2_ref, r_ref, o_ref):
    r = jnp.maximum(ea_ref[...] @ w1_ref[...] + b1_ref[...], 0.0)
    ew = jnp.dot(r.astype(jnp.bfloat16), w2_ref[...],
                 preferred_element_type=_f32) + b2_ref[...]
    xb = jnp.dot(xj_ref[...].astype(jnp.bfloat16), r_ref[...],
                 preferred_element_type=_f32)
    p = xb * ew
    p = p[:, :512] + p[:, 512:]
    p = p[:, :256] + p[:, 256:]
    p = p[:, :128] + p[:, 128:]
    p = p[:, :64] + p[:, 64:]
    o_ref[...] = p[:, :32] + p[:, 32:]


def _tc_edge(xj, eap, We1p, be1r, We2T, be2r, Rm):
    return pl.pallas_call(
        _edge_body,
        grid=(EP // EB,),
        in_specs=[
            pl.BlockSpec((EB, H), lambda i: (i, 0)),
            pl.BlockSpec((EB, 8), lambda i: (i, 0)),
            _full((8, 128)),
            _full((1, 128)),
            _full((128, H * H)),
            _full((1, H * H)),
            _full((H, H * H)),
        ],
        out_specs=pl.BlockSpec((EB, H), lambda i: (i, 0)),
        out_shape=jax.ShapeDtypeStruct((EP, H), _f32),
    )(xj, eap, We1p, be1r, We2T, be2r, Rm)


def _node_body(ap_ref, dp_ref, s_ref, wr_ref, bc_ref, wi_ref, bi_ref, wh_ref, bh_ref, o_ref):
    deg = jnp.maximum(dp_ref[0, :, 0:1] + dp_ref[1, :, 0:1], 1.0)
    agg = (ap_ref[0] + ap_ref[1]) / deg
    s = s_ref[...]
    m = jnp.maximum(agg + s @ wr_ref[...] + bc_ref[...], 0.0)
    gi = m @ wi_ref[...] + bi_ref[...]
    gh = s @ wh_ref[...] + bh_ref[...]
    r = jax.nn.sigmoid(gi[:, 0:H] + gh[:, 0:H])
    z = jax.nn.sigmoid(gi[:, H:2 * H] + gh[:, H:2 * H])
    n = jnp.tanh(gi[:, 2 * H:3 * H] + r * gh[:, 2 * H:3 * H])
    o_ref[...] = (1.0 - z) * n + z * s


def _tc_node(aggp, degp, s, WrootM, bconvr, WihT, bihr, WhhT, bhhr):
    return pl.pallas_call(
        _node_body,
        grid=(NP_ // NB,),
        in_specs=[
            pl.BlockSpec((2, NB, H), lambda i: (0, i, 0)),
            pl.BlockSpec((2, NB, 16), lambda i: (0, i, 0)),
            pl.BlockSpec((NB, H), lambda i: (i, 0)),
            _full((H, H)),
            _full((1, H)),
            _full((H, 3 * H)),
            _full((1, 3 * H)),
            _full((H, 3 * H)),
            _full((1, 3 * H)),
        ],
        out_specs=pl.BlockSpec((NB, H), lambda i: (i, 0)),
        out_shape=jax.ShapeDtypeStruct((NP_, H), _f32),
    )(aggp, degp, s, WrootM, bconvr, WihT, bihr, WhhT, bhhr)


def _ne_body(s_ref, w_ref, b_ref, o_ref):
    o_ref[...] = s_ref[...] @ w_ref[...] + b_ref[...]


def _tc_ne(s, W1T, b1r):
    return pl.pallas_call(
        _ne_body,
        grid=(NP_ // NB,),
        in_specs=[
            pl.BlockSpec((NB, H), lambda i: (i, 0)),
            _full((H, D)),
            _full((1, D)),
        ],
        out_specs=pl.BlockSpec((NB, D), lambda i: (i, 0)),
        out_shape=jax.ShapeDtypeStruct((NP_, D), _f32),
    )(s, W1T, b1r)


def _lstm_body(q_ref, rv_ref, hs_ref, cs_ref, wi_ref, bi_ref, wh_ref, bh_ref, ho_ref, co_ref):
    rvec = rv_ref[:, 0:D] / (rv_ref[:, D:D + 1] + 1e-16)
    q_star = jnp.concatenate([q_ref[...], rvec], axis=1)
    gates = q_star @ wi_ref[...] + bi_ref[...] + hs_ref[...] @ wh_ref[...] + bh_ref[...]
    gi = gates[:, 0:D]
    gf = gates[:, D:2 * D]
    gg = gates[:, 2 * D:3 * D]
    go = gates[:, 3 * D:4 * D]
    cs = jax.nn.sigmoid(gf) * cs_ref[...] + jax.nn.sigmoid(gi) * jnp.tanh(gg)
    ho_ref[...] = jax.nn.sigmoid(go) * jnp.tanh(cs)
    co_ref[...] = cs


def _tc_lstm(qprev, rvacc, hs, cs, WihsT, bihsr, WhhsT, bhhsr):
    return pl.pallas_call(
        _lstm_body,
        out_shape=(
            jax.ShapeDtypeStruct((B, D), _f32),
            jax.ShapeDtypeStruct((B, D), _f32),
        ),
    )(qprev, rvacc, hs, cs, WihsT, bihsr, WhhsT, bhhsr)


def _pass1_body(ne_ref, b_ref, q_ref, e_ref, mx_ref):
    pid = pl.program_id(0)
    cols = lax.broadcasted_iota(jnp.int32, (NB, B), 1)
    ohb = b_ref[...] == cols
    oh = ohb.astype(_f32)
    qg = oh @ q_ref[...]
    e = jnp.sum(ne_ref[...] * qg, axis=1, keepdims=True)
    e_ref[...] = e
    part = jnp.max(jnp.where(ohb, e, -1e30), axis=0, keepdims=True)

    @pl.when(pid == 0)
    def _():
        mx_ref[...] = jnp.full((1, B), -1e30, _f32)

    mx_ref[...] = jnp.maximum(mx_ref[...], part)


def _tc_pass1(nep, batchc, q):
    return pl.pallas_call(
        _pass1_body,
        grid=(NP_ // NB,),
        in_specs=[
            pl.BlockSpec((NB, D), lambda i: (i, 0)),
            pl.BlockSpec((NB, 1), lambda i: (i, 0)),
            _full((B, D)),
        ],
        out_specs=(
            pl.BlockSpec((NB, 1), lambda i: (i, 0)),
            pl.BlockSpec((1, B), lambda i: (0, 0)),
        ),
        out_shape=(
            jax.ShapeDtypeStruct((NP_, 1), _f32),
            jax.ShapeDtypeStruct((1, B), _f32),
        ),
    )(nep, batchc, q)


def _pass2_body(ne_ref, b_ref, e_ref, mx_ref, rv_ref):
    pid = pl.program_id(0)
    cols = lax.broadcasted_iota(jnp.int32, (NB, B), 1)
    oh = (b_ref[...] == cols).astype(_f32)
    mxg = jnp.sum(oh * mx_ref[...], axis=1, keepdims=True)
    a = jnp.exp(e_ref[...] - mxg)
    oa = oh * a
    ne_aug = jnp.concatenate([ne_ref[...], jnp.ones((NB, 1), _f32)], axis=1)
    part = lax.dot_general(oa, ne_aug, (((0,), (0,)), ((), ())),
                           preferred_element_type=_f32)

    @pl.when(pid == 0)
    def _():
        rv_ref[...] = jnp.zeros((B, D + 1), _f32)

    rv_ref[...] = rv_ref[...] + part


def _tc_pass2(nep, batchc, e_col, emax):
    return pl.pallas_call(
        _pass2_body,
        grid=(NP_ // NB,),
        in_specs=[
            pl.BlockSpec((NB, D), lambda i: (i, 0)),
            pl.BlockSpec((NB, 1), lambda i: (i, 0)),
            pl.BlockSpec((NB, 1), lambda i: (i, 0)),
            _full((1, B)),
        ],
        out_specs=pl.BlockSpec((B, D + 1), lambda i: (0, 0)),
        out_shape=jax.ShapeDtypeStruct((B, D + 1), _f32),
    )(nep, batchc, e_col, emax)


def _head_body(q_ref, rv_ref, w2_ref, b2_ref, w3_ref, b3_ref, ge_ref, pr_ref):
    rvec = rv_ref[:, 0:D] / (rv_ref[:, D:D + 1] + 1e-16)
    q_star = jnp.concatenate([q_ref[...], rvec], axis=1)
    ge = q_star @ w2_ref[...] + b2_ref[...]
    ge_ref[...] = ge
    pr_ref[...] = ge @ w3_ref[...] + b3_ref[...]


def _tc_head(qprev, rvacc, W2T, b2r, W3T, b3r):
    return pl.pallas_call(
        _head_body,
        out_shape=(
            jax.ShapeDtypeStruct((B, D), _f32),
            jax.ShapeDtypeStruct((B, 1), _f32),
        ),
    )(qprev, rvacc, W2T, b2r, W3T, b3r)


def kernel(x, edge_index, edge_attr, batch, W0, b0, We1, be1, We2, be2,
           Wroot, bconv, Wih, Whh, bih, bhh, Wih_s, Whh_s, bih_s, bhh_s,
           W1, b1, W2, b2, W3, b3):
    xp = jnp.pad(x, ((0, NP_ - N), (0, 0)))
    src2d = jnp.pad(edge_index[0], (0, EP - E)).reshape(EP // 128, 128)
    dst2d = jnp.pad(edge_index[1], (0, EP - E),
                    constant_values=NP_ - 1).reshape(EP // 128, 128)
    eap = jnp.pad(edge_attr, ((0, EP - E), (0, 5)))
    batchc = jnp.pad(batch, (0, NP_ - N), constant_values=B).reshape(NP_, 1)

    W0T = W0.T
    b0r = b0.reshape(1, -1)
    We1p = jnp.pad(We1.T, ((0, 5), (0, 0)))
    be1r = be1.reshape(1, -1)
    We2T = We2.T.astype(jnp.bfloat16)
    be2r = be2.reshape(1, -1)
    bconvr = bconv.reshape(1, -1)
    WihT = Wih.T
    bihr = bih.reshape(1, -1)
    WhhT = Whh.T
    bhhr = bhh.reshape(1, -1)
    WihsT = Wih_s.T
    bihsr = bih_s.reshape(1, -1)
    WhhsT = Whh_s.T
    bhhsr = bhh_s.reshape(1, -1)
    W1T = W1.T
    b1r = b1.reshape(1, -1)
    W2T = W2.T
    b2r = b2.reshape(1, -1)
    W3T = W3.T
    b3r = b3.reshape(1, -1)

    Rm = jnp.asarray(np.kron(np.eye(H, dtype=np.float32), np.ones((1, H), np.float32))).astype(jnp.bfloat16)
    ones16 = jnp.ones((128, 16), _f32)
    zeros16 = jnp.zeros((NPS, 16), _f32)
    zeros32 = jnp.zeros((NPS, H), _f32)

    s = _tc_out0(xp, W0T, b0r)
    degp = _sc_deg(dst2d, ones16, zeros16)
    for _ in range(3):
        xj = _sc_gather(src2d, s)
        msg = _tc_edge(xj, eap, We1p, be1r, We2T, be2r, Rm)
        aggp = _sc_scatter(dst2d, msg, zeros32)
        s = _tc_node(aggp, degp, s, Wroot, bconvr, WihT, bihr, WhhT, bhhr)

    nep = _tc_ne(s, W1T, b1r)
    hs = jnp.zeros((B, D), _f32)
    cs = jnp.zeros((B, D), _f32)
    qprev = jnp.zeros((B, D), _f32)
    rvacc = jnp.zeros((B, D + 1), _f32)
    for _ in range(3):
        hs, cs = _tc_lstm(qprev, rvacc, hs, cs, WihsT, bihsr, WhhsT, bhhsr)
        qprev = hs
        e_col, emax = _tc_pass1(nep, batchc, hs)
        rvacc = _tc_pass2(nep, batchc, e_col, emax)

    ge, pred = _tc_head(qprev, rvacc, W2T, b2r, W3T, b3r)
    return pred.reshape(-1), ge, nep[:N]


# speedup vs baseline: 2.6478x; 1.0321x over previous
import functools

import jax
import jax.numpy as jnp
import numpy as np
from jax import lax
from jax.experimental import pallas as pl
from jax.experimental.pallas import tpu as pltpu
from jax.experimental.pallas import tpu_sc as plsc

N = 10000
E = 160000
NF = 128
H = 32
D = 32
B = 512

NP_ = 10240
EP = 163840
NW = 32
EPW = EP // NW
CPW = EPW // 128
NB = 256
EB = 1024
NPS = NP_ // 16
EPH = EP // 2
EWH = EPH // NW
CPWH = EWH // 128

_f32 = jnp.float32


def _sc_mesh():
    return plsc.VectorSubcoreMesh(core_axis_name="c", subcore_axis_name="s")


_SC_PARAMS = pltpu.CompilerParams(use_tc_tiling_on_sc=False)


def _sc_gather(src2d, table):
    @functools.partial(
        pl.kernel,
        out_type=jax.ShapeDtypeStruct((EPH, H), _f32),
        mesh=_sc_mesh(),
        compiler_params=_SC_PARAMS,
        scratch_types=[
            pltpu.VMEM((CPWH, 128), jnp.int32),
            pltpu.VMEM((EWH, H), _f32),
            pltpu.SemaphoreType.DMA,
        ],
    )
    def k(src_hbm, tab_hbm, out_hbm, idx_v, rows_v, sem):
        c = lax.axis_index("c")
        s = lax.axis_index("s")
        wid = c * 16 + s
        pltpu.sync_copy(src_hbm.at[pl.ds(wid * CPWH, CPWH), :], idx_v)
        cps = [
            pltpu.async_copy(
                tab_hbm.at[idx_v.at[j]],
                rows_v.at[pl.ds(j * 128, 128), :],
                sem,
            )
            for j in range(CPWH)
        ]
        for cp in cps:
            cp.wait()
        pltpu.sync_copy(rows_v, out_hbm.at[pl.ds(wid * EWH, EWH), :])

    return k(src2d, table)


def _sc_scatter(dst2d, msg, init):
    @functools.partial(
        pl.kernel,
        out_type=jax.ShapeDtypeStruct((2, NP_, H), _f32),
        mesh=_sc_mesh(),
        compiler_params=_SC_PARAMS,
        scratch_types=[
            pltpu.VMEM((CPWH, 128), jnp.int32),
            pltpu.VMEM((1280, H), _f32),
            pltpu.VMEM((1280, H), _f32),
            pltpu.VMEM_SHARED((NP_, H), _f32),
            pltpu.SemaphoreType.DMA,
            pltpu.SemaphoreType.DMA,
        ],
    )
    def k(dst_hbm, msg_hbm, init_hbm, out_hbm, idx_v, buf0_v, buf1_v, acc_sh,
          sem_l, sem_s):
        c = lax.axis_index("c")
        s = lax.axis_index("s")
        wid = c * 16 + s
        pltpu.sync_copy(init_hbm.at[c, pl.ds(s * NPS, NPS), :],
                        buf0_v.at[pl.ds(0, NPS), :])
        pltpu.sync_copy(buf0_v.at[pl.ds(0, NPS), :], acc_sh.at[pl.ds(s * NPS, NPS), :])
        plsc.subcore_barrier()
        pltpu.sync_copy(dst_hbm.at[pl.ds(wid * CPWH, CPWH), :], idx_v)
        bufs = (buf0_v, buf1_v)
        scat = [[], []]
        loads = [None, None]
        loads[0] = pltpu.async_copy(
            msg_hbm.at[pl.ds(wid * EWH, 1280), :], buf0_v, sem_l)
        for q in range(2):
            b = q % 2
            loads[b].wait()
            scat[b] = [
                pltpu.async_copy(
                    bufs[b].at[pl.ds(j * 128, 128), :],
                    acc_sh.at[idx_v.at[q * 10 + j]],
                    sem_s,
                    add=True,
                )
                for j in range(10)
            ]
            if q < 1:
                nb = (q + 1) % 2
                loads[nb] = pltpu.async_copy(
                    msg_hbm.at[pl.ds(wid * EWH + (q + 1) * 1280, 1280), :],
                    bufs[nb], sem_l)
        for b in range(2):
            for cp in scat[b]:
                cp.wait()
        plsc.subcore_barrier()
        pltpu.sync_copy(acc_sh.at[pl.ds(s * NPS, NPS), :], buf0_v.at[pl.ds(0, NPS), :])
        pltpu.sync_copy(buf0_v.at[pl.ds(0, NPS), :], out_hbm.at[c, pl.ds(s * NPS, NPS), :])

    return k(dst2d, msg, init)


def _sc_deg(dst2d, ones16, zeros16):
    @functools.partial(
        pl.kernel,
        out_type=jax.ShapeDtypeStruct((2, NP_, 16), _f32),
        mesh=_sc_mesh(),
        compiler_params=_SC_PARAMS,
        scratch_types=[
            pltpu.VMEM((CPW, 128), jnp.int32),
            pltpu.VMEM((128, 16), _f32),
            pltpu.VMEM((NPS, 16), _f32),
            pltpu.VMEM_SHARED((NP_, 16), _f32),
        ],
    )
    def k(dst_hbm, ones_hbm, z_hbm, out_hbm, idx_v, ones_v, buf_v, acc_sh):
        c = lax.axis_index("c")
        s = lax.axis_index("s")
        wid = c * 16 + s
        pltpu.sync_copy(z_hbm, buf_v)
        pltpu.sync_copy(buf_v, acc_sh.at[pl.ds(s * NPS, NPS), :])
        plsc.subcore_barrier()
        pltpu.sync_copy(ones_hbm, ones_v)
        pltpu.sync_copy(dst_hbm.at[pl.ds(wid * CPW, CPW), :], idx_v)
        for j in range(CPW):
            pltpu.sync_copy(ones_v, acc_sh.at[idx_v.at[j]], add=True)
        plsc.subcore_barrier()
        pltpu.sync_copy(acc_sh.at[pl.ds(s * NPS, NPS), :], buf_v)
        pltpu.sync_copy(buf_v, out_hbm.at[c, pl.ds(s * NPS, NPS), :])

    return k(dst2d, ones16, zeros16)


def _full(shape):
    nd = len(shape)
    return pl.BlockSpec(shape, lambda i: (0,) * nd)


def _out0_body(x_ref, w_ref, b_ref, o_ref):
    o_ref[...] = jnp.maximum(x_ref[...] @ w_ref[...] + b_ref[...], 0.0)


def _tc_out0(xp, W0T, b0r):
    return pl.pallas_call(
        _out0_body,
        grid=(NP_ // NB,),
        in_specs=[
            pl.BlockSpec((NB, NF), lambda i: (i, 0)),
            _full((NF, H)),
            _full((1, H)),
        ],
        out_specs=pl.BlockSpec((NB, H), lambda i: (i, 0)),
        out_shape=jax.ShapeDtypeStruct((NP_, H), _f32),
    )(xp, W0T, b0r)


def _edge_body(xj_ref, ea_ref, w1_ref, b1_ref, w2_ref, b2_ref, r_ref, o_ref):
    r = jnp.maximum(ea_ref[...] @ w1_ref[...] + b1_ref[...], 0.0)
    ew = jnp.dot(r, w2_ref[...], preferred_element_type=_f32) + b2_ref[...]
    xb = jnp.dot(xj_ref[...], r_ref[...], preferred_element_type=_f32)
    p = xb * ew
    p = p[:, :512] + p[:, 512:]
    p = p[:, :256] + p[:, 256:]
    p = p[:, :128] + p[:, 128:]
    p = p[:, :64] + p[:, 64:]
    o_ref[...] = p[:, :32] + p[:, 32:]


def _tc_edge(xj, eap, We1p, be1r, We2T, be2r, Rm):
    return pl.pallas_call(
        _edge_body,
        grid=(EPH // EB,),
        in_specs=[
            pl.BlockSpec((EB, H), lambda i: (i, 0)),
            pl.BlockSpec((EB, 8), lambda i: (i, 0)),
            _full((8, 128)),
            _full((1, 128)),
            _full((128, H * H)),
            _full((1, H * H)),
            _full((H, H * H)),
        ],
        out_specs=pl.BlockSpec((EB, H), lambda i: (i, 0)),
        out_shape=jax.ShapeDtypeStruct((EPH, H), _f32),
    )(xj, eap, We1p, be1r, We2T, be2r, Rm)


def _node_body(ap_ref, dp_ref, s_ref, wr_ref, bc_ref, wi_ref, bi_ref, wh_ref, bh_ref, o_ref):
    deg = jnp.maximum(dp_ref[0, :, 0:1] + dp_ref[1, :, 0:1], 1.0)
    agg = (ap_ref[0] + ap_ref[1]) / deg
    s = s_ref[...]
    m = jnp.maximum(agg + s @ wr_ref[...] + bc_ref[...], 0.0)
    gi = m @ wi_ref[...] + bi_ref[...]
    gh = s @ wh_ref[...] + bh_ref[...]
    r = jax.nn.sigmoid(gi[:, 0:H] + gh[:, 0:H])
    z = jax.nn.sigmoid(gi[:, H:2 * H] + gh[:, H:2 * H])
    n = jnp.tanh(gi[:, 2 * H:3 * H] + r * gh[:, 2 * H:3 * H])
    o_ref[...] = (1.0 - z) * n + z * s


def _tc_node(aggp, degp, s, WrootM, bconvr, WihT, bihr, WhhT, bhhr):
    return pl.pallas_call(
        _node_body,
        grid=(NP_ // NB,),
        in_specs=[
            pl.BlockSpec((2, NB, H), lambda i: (0, i, 0)),
            pl.BlockSpec((2, NB, 16), lambda i: (0, i, 0)),
            pl.BlockSpec((NB, H), lambda i: (i, 0)),
            _full((H, H)),
            _full((1, H)),
            _full((H, 3 * H)),
            _full((1, 3 * H)),
            _full((H, 3 * H)),
            _full((1, 3 * H)),
        ],
        out_specs=pl.BlockSpec((NB, H), lambda i: (i, 0)),
        out_shape=jax.ShapeDtypeStruct((NP_, H), _f32),
    )(aggp, degp, s, WrootM, bconvr, WihT, bihr, WhhT, bhhr)


def _ne_body(s_ref, w_ref, b_ref, o_ref):
    o_ref[...] = s_ref[...] @ w_ref[...] + b_ref[...]


def _tc_ne(s, W1T, b1r):
    return pl.pallas_call(
        _ne_body,
        grid=(NP_ // NB,),
        in_specs=[
            pl.BlockSpec((NB, H), lambda i: (i, 0)),
            _full((H, D)),
            _full((1, D)),
        ],
        out_specs=pl.BlockSpec((NB, D), lambda i: (i, 0)),
        out_shape=jax.ShapeDtypeStruct((NP_, D), _f32),
    )(s, W1T, b1r)


def _lstm_body(q_ref, rv_ref, hs_ref, cs_ref, wi_ref, bi_ref, wh_ref, bh_ref, ho_ref, co_ref):
    rvec = rv_ref[:, 0:D] / (rv_ref[:, D:D + 1] + 1e-16)
    q_star = jnp.concatenate([q_ref[...], rvec], axis=1)
    gates = q_star @ wi_ref[...] + bi_ref[...] + hs_ref[...] @ wh_ref[...] + bh_ref[...]
    gi = gates[:, 0:D]
    gf = gates[:, D:2 * D]
    gg = gates[:, 2 * D:3 * D]
    go = gates[:, 3 * D:4 * D]
    cs = jax.nn.sigmoid(gf) * cs_ref[...] + jax.nn.sigmoid(gi) * jnp.tanh(gg)
    ho_ref[...] = jax.nn.sigmoid(go) * jnp.tanh(cs)
    co_ref[...] = cs


def _tc_lstm(qprev, rvacc, hs, cs, WihsT, bihsr, WhhsT, bhhsr):
    return pl.pallas_call(
        _lstm_body,
        out_shape=(
            jax.ShapeDtypeStruct((B, D), _f32),
            jax.ShapeDtypeStruct((B, D), _f32),
        ),
    )(qprev, rvacc, hs, cs, WihsT, bihsr, WhhsT, bhhsr)


def _pass1_body(ne_ref, b_ref, q_ref, e_ref, mx_ref):
    pid = pl.program_id(0)
    cols = lax.broadcasted_iota(jnp.int32, (NB, B), 1)
    ohb = b_ref[...] == cols
    oh = ohb.astype(_f32)
    qg = oh @ q_ref[...]
    e = jnp.sum(ne_ref[...] * qg, axis=1, keepdims=True)
    e_ref[...] = e
    part = jnp.max(jnp.where(ohb, e, -1e30), axis=0, keepdims=True)

    @pl.when(pid == 0)
    def _():
        mx_ref[...] = jnp.full((1, B), -1e30, _f32)

    mx_ref[...] = jnp.maximum(mx_ref[...], part)


def _tc_pass1(nep, batchc, q):
    return pl.pallas_call(
        _pass1_body,
        grid=(NP_ // NB,),
        in_specs=[
            pl.BlockSpec((NB, D), lambda i: (i, 0)),
            pl.BlockSpec((NB, 1), lambda i: (i, 0)),
            _full((B, D)),
        ],
        out_specs=(
            pl.BlockSpec((NB, 1), lambda i: (i, 0)),
            pl.BlockSpec((1, B), lambda i: (0, 0)),
        ),
        out_shape=(
            jax.ShapeDtypeStruct((NP_, 1), _f32),
            jax.ShapeDtypeStruct((1, B), _f32),
        ),
    )(nep, batchc, q)


def _pass2_body(ne_ref, b_ref, e_ref, mx_ref, rv_ref):
    pid = pl.program_id(0)
    cols = lax.broadcasted_iota(jnp.int32, (NB, B), 1)
    oh = (b_ref[...] == cols).astype(_f32)
    mxg = jnp.sum(oh * mx_ref[...], axis=1, keepdims=True)
    a = jnp.exp(e_ref[...] - mxg)
    oa = oh * a
    ne_aug = jnp.concatenate([ne_ref[...], jnp.ones((NB, 1), _f32)], axis=1)
    part = lax.dot_general(oa, ne_aug, (((0,), (0,)), ((), ())),
                           preferred_element_type=_f32)

    @pl.when(pid == 0)
    def _():
        rv_ref[...] = jnp.zeros((B, D + 1), _f32)

    rv_ref[...] = rv_ref[...] + part


def _tc_pass2(nep, batchc, e_col, emax):
    return pl.pallas_call(
        _pass2_body,
        grid=(NP_ // NB,),
        in_specs=[
            pl.BlockSpec((NB, D), lambda i: (i, 0)),
            pl.BlockSpec((NB, 1), lambda i: (i, 0)),
            pl.BlockSpec((NB, 1), lambda i: (i, 0)),
            _full((1, B)),
        ],
        out_specs=pl.BlockSpec((B, D + 1), lambda i: (0, 0)),
        out_shape=jax.ShapeDtypeStruct((B, D + 1), _f32),
    )(nep, batchc, e_col, emax)


def _head_body(q_ref, rv_ref, w2_ref, b2_ref, w3_ref, b3_ref, ge_ref, pr_ref):
    rvec = rv_ref[:, 0:D] / (rv_ref[:, D:D + 1] + 1e-16)
    q_star = jnp.concatenate([q_ref[...], rvec], axis=1)
    ge = q_star @ w2_ref[...] + b2_ref[...]
    ge_ref[...] = ge
    pr_ref[...] = ge @ w3_ref[...] + b3_ref[...]


def _tc_head(qprev, rvacc, W2T, b2r, W3T, b3r):
    return pl.pallas_call(
        _head_body,
        out_shape=(
            jax.ShapeDtypeStruct((B, D), _f32),
            jax.ShapeDtypeStruct((B, 1), _f32),
        ),
    )(qprev, rvacc, W2T, b2r, W3T, b3r)


def kernel(x, edge_index, edge_attr, batch, W0, b0, We1, be1, We2, be2,
           Wroot, bconv, Wih, Whh, bih, bhh, Wih_s, Whh_s, bih_s, bhh_s,
           W1, b1, W2, b2, W3, b3):
    xp = jnp.pad(x, ((0, NP_ - N), (0, 0)))
    src2d = jnp.pad(edge_index[0], (0, EP - E)).reshape(EP // 128, 128)
    dst2d = jnp.pad(edge_index[1], (0, EP - E),
                    constant_values=NP_ - 1).reshape(EP // 128, 128)
    eap = jnp.pad(edge_attr, ((0, EP - E), (0, 5)))
    batchc = jnp.pad(batch, (0, NP_ - N), constant_values=B).reshape(NP_, 1)

    W0T = W0.T
    b0r = b0.reshape(1, -1)
    We1p = jnp.pad(We1.T, ((0, 5), (0, 0)))
    be1r = be1.reshape(1, -1)
    We2T = We2.T
    be2r = be2.reshape(1, -1)
    bconvr = bconv.reshape(1, -1)
    WihT = Wih.T
    bihr = bih.reshape(1, -1)
    WhhT = Whh.T
    bhhr = bhh.reshape(1, -1)
    WihsT = Wih_s.T
    bihsr = bih_s.reshape(1, -1)
    WhhsT = Whh_s.T
    bhhsr = bhh_s.reshape(1, -1)
    W1T = W1.T
    b1r = b1.reshape(1, -1)
    W2T = W2.T
    b2r = b2.reshape(1, -1)
    W3T = W3.T
    b3r = b3.reshape(1, -1)

    Rm = jnp.asarray(np.kron(np.eye(H, dtype=np.float32), np.ones((1, H), np.float32)))
    ones16 = jnp.ones((128, 16), _f32)
    zeros16 = jnp.zeros((NPS, 16), _f32)

    s = _tc_out0(xp, W0T, b0r)
    degp = _sc_deg(dst2d, ones16, zeros16)
    src_h = (src2d[: EP // 256], src2d[EP // 256:])
    dst_h = (dst2d[: EP // 256], dst2d[EP // 256:])
    ea_h = (eap[:EPH], eap[EPH:])
    aggz = jnp.zeros((2, NP_, H), _f32)
    for _ in range(3):
        xj0 = _sc_gather(src_h[0], s)
        msg0 = _tc_edge(xj0, ea_h[0], We1p, be1r, We2T, be2r, Rm)
        xj1 = _sc_gather(src_h[1], s)
        p0 = _sc_scatter(dst_h[0], msg0, aggz)
        msg1 = _tc_edge(xj1, ea_h[1], We1p, be1r, We2T, be2r, Rm)
        aggp = _sc_scatter(dst_h[1], msg1, p0)
        s = _tc_node(aggp, degp, s, Wroot, bconvr, WihT, bihr, WhhT, bhhr)

    nep = _tc_ne(s, W1T, b1r)
    hs = jnp.zeros((B, D), _f32)
    cs = jnp.zeros((B, D), _f32)
    qprev = jnp.zeros((B, D), _f32)
    rvacc = jnp.zeros((B, D + 1), _f32)
    for _ in range(3):
        hs, cs = _tc_lstm(qprev, rvacc, hs, cs, WihsT, bihsr, WhhsT, bhhsr)
        qprev = hs
        e_col, emax = _tc_pass1(nep, batchc, hs)
        rvacc = _tc_pass2(nep, batchc, e_col, emax)

    ge, pred = _tc_head(qprev, rvacc, W2T, b2r, W3T, b3r)
    return pred.reshape(-1), ge, nep[:N]
